# Initial kernel scaffold; baseline (speedup 1.0000x reference)
import jax
import jax.numpy as jnp
from jax.experimental import pallas as pl


def kernel(q, k, v, block_table):
    raise NotImplementedError("write your pallas kernel here")


# speedup vs baseline: 6.6631x; 6.6631x over previous
import functools
import math

import jax
import jax.numpy as jnp
from jax.experimental import pallas as pl
from jax.experimental.pallas import tpu as pltpu


def _attn_body(q_ref, k_ref, v_ref, o_ref, *, S, Qn, G, D):
    qm = q_ref[0, 0]
    km = k_ref[0]
    vm = v_ref[0]

    ks = jnp.maximum(jnp.max(jnp.abs(km), axis=1, keepdims=True) / 7.0, 1e-8)
    kq = jnp.clip(jnp.round(km / ks), -8.0, 7.0) * ks
    vs = jnp.maximum(jnp.max(jnp.abs(vm), axis=1, keepdims=True) / 7.0, 1e-8)
    vq = jnp.clip(jnp.round(vm / vs), -8.0, 7.0) * vs

    scores = jax.lax.dot_general(
        qm, kq, (((1,), (1,)), ((), ())),
        preferred_element_type=jnp.float32) * (1.0 / math.sqrt(D))

    rows = jax.lax.broadcasted_iota(jnp.int32, (Qn * G, S), 0)
    cols = jax.lax.broadcasted_iota(jnp.int32, (Qn * G, S), 1)
    qpos = (S - Qn) + rows // G
    scores = jnp.where(cols <= qpos, scores, jnp.float32(-1e30))

    m = jnp.max(scores, axis=1, keepdims=True)
    p = jnp.exp(scores - m)
    l = jnp.sum(p, axis=1, keepdims=True)
    o = jax.lax.dot_general(
        p, vq, (((1,), (0,)), ((), ())),
        preferred_element_type=jnp.float32)
    o_ref[0, 0] = o / l


def kernel(q, k, v, block_table):
    B, Qn, H, D = q.shape
    _, S, KVH, _ = k.shape
    G = H // KVH

    qg = (q.reshape(B, Qn, KVH, G, D)
           .transpose(0, 2, 1, 3, 4)
           .reshape(B, KVH, Qn * G, D))

    kf = k.reshape(B, S, KVH * D)
    vf = v.reshape(B, S, KVH * D)

    out = pl.pallas_call(
        functools.partial(_attn_body, S=S, Qn=Qn, G=G, D=D),
        grid=(B, KVH),
        in_specs=[
            pl.BlockSpec((1, 1, Qn * G, D), lambda b, h: (b, h, 0, 0)),
            pl.BlockSpec((1, S, D), lambda b, h: (b, 0, h)),
            pl.BlockSpec((1, S, D), lambda b, h: (b, 0, h)),
        ],
        out_specs=pl.BlockSpec((1, 1, Qn * G, D), lambda b, h: (b, h, 0, 0)),
        out_shape=jax.ShapeDtypeStruct((B, KVH, Qn * G, D), jnp.float32),
        compiler_params=pltpu.CompilerParams(
            dimension_semantics=("parallel", "parallel"),
        ),
    )(qg, kf, vf)

    return (out.reshape(B, KVH, Qn, G, D)
               .transpose(0, 2, 1, 3, 4)
               .reshape(B, Qn, H, D))
